# Initial kernel scaffold; baseline (speedup 1.0000x reference)
import jax
import jax.numpy as jnp
from jax.experimental import pallas as pl


def kernel(x, edge_index, W1, b1, W2, b2):
    raise NotImplementedError("write your pallas kernel here")


# speedup vs baseline: 3.0998x; 3.0998x over previous
import functools

import jax
import jax.numpy as jnp
from jax import lax
from jax.experimental import pallas as pl
from jax.experimental.pallas import tpu as pltpu
from jax.experimental.pallas import tpu_sc as plsc

_N = 10000
_E = 320000
_D = 128

_NW = 32
_FPW = _D // _NW
_CHUNK = 3200
_NCHUNKS = _E // _CHUNK
_GROUPS = _CHUNK // 16
_BN = 1000


def _mm1_body(x_ref, w_ref, b_ref, a_out, bv_out):
    xb = x_ref[...]
    w = w_ref[...]
    wt = w[:_D]
    wb = w[_D:]
    dn = (((0,), (1,)), ((), ()))
    a = lax.dot_general(wt - wb, xb, dn, preferred_element_type=jnp.float32)
    a_out[...] = a + b_ref[...]
    bv_out[...] = lax.dot_general(wb, xb, dn, preferred_element_type=jnp.float32)


def _layer1_matmuls(x, w1, b1col):
    return pl.pallas_call(
        _mm1_body,
        out_shape=[
            jax.ShapeDtypeStruct((_D, _N), jnp.float32),
            jax.ShapeDtypeStruct((_D, _N), jnp.float32),
        ],
    )(x, w1, b1col)


def _mm2_body(a_ref, m_ref, w_ref, b_ref, a_out, bv_out):
    t = a_ref[...] + m_ref[...]
    h = jnp.maximum(jnp.where(jnp.isfinite(t), t, 0.0), 0.0)
    w = w_ref[...]
    wt = w[:_D]
    wb = w[_D:]
    dn = (((0,), (0,)), ((), ()))
    a2 = lax.dot_general(wt - wb, h, dn, preferred_element_type=jnp.float32)
    a_out[...] = a2 + b_ref[...]
    bv_out[...] = lax.dot_general(wb, h, dn, preferred_element_type=jnp.float32)


def _layer2_matmuls(a1t, m1t, w2, b2col):
    return pl.pallas_call(
        _mm2_body,
        out_shape=[
            jax.ShapeDtypeStruct((_D, _N), jnp.float32),
            jax.ShapeDtypeStruct((_D, _N), jnp.float32),
        ],
    )(a1t, m1t, w2, b2col)


def _fin_body(a_ref, m_ref, o_ref):
    t = a_ref[...] + m_ref[...]
    o = jnp.where(jnp.isfinite(t), t, 0.0)
    mx = jnp.max(o, axis=0, keepdims=True)
    lse = jnp.log(jnp.sum(jnp.exp(o - mx), axis=0, keepdims=True)) + mx
    o_ref[...] = (o - lse).T


def _finalize(a2t, m2t):
    return pl.pallas_call(
        _fin_body,
        out_shape=jax.ShapeDtypeStruct((_N, _D), jnp.float32),
    )(a2t, m2t)


def _sc_segment_max(bt_flat, src, dst):
    mesh = plsc.VectorSubcoreMesh(core_axis_name="c", subcore_axis_name="s", num_cores=2, num_subcores=16)

    @functools.partial(
        pl.kernel,
        out_type=jax.ShapeDtypeStruct((_D * _N,), jnp.float32),
        mesh=mesh,
        compiler_params=pltpu.CompilerParams(needs_layout_passes=False),
        scratch_types=[
            pltpu.VMEM((_FPW * _N,), jnp.float32),
            pltpu.VMEM((_FPW * _N,), jnp.float32),
            pltpu.VMEM((_N,), jnp.int32),
            pltpu.VMEM((_CHUNK,), jnp.int32),
            pltpu.VMEM((_CHUNK,), jnp.int32),
            pltpu.VMEM((_CHUNK,), jnp.int32),
            pltpu.VMEM((_CHUNK,), jnp.int32),
            pltpu.SemaphoreType.DMA,
            pltpu.SemaphoreType.DMA,
            pltpu.SemaphoreType.DMA,
            pltpu.SemaphoreType.DMA,
        ],
    )
    def k(bt_hbm, src_hbm, dst_hbm, out_hbm,
          b_v, m_v, scr_v, s0, s1, d0, d1, sem_s0, sem_s1, sem_d0, sem_d1):
        wid = lax.axis_index("s") * 2 + lax.axis_index("c")
        base = wid * (_FPW * _N)

        pltpu.sync_copy(bt_hbm.at[pl.ds(base, _FPW * _N)], b_v)

        neg = jnp.full((16,), -jnp.inf, jnp.float32)

        def init_body(i, c):
            m_v[pl.ds(i * 16, 16)] = neg
            return c
        lax.fori_loop(0, _FPW * _N // 16, init_body, 0)

        lane = lax.iota(jnp.int32, 16)

        def process(sbuf, dbuf):
            def group(j, c):
                s = sbuf[pl.ds(j * 16, 16)]
                d = dbuf[pl.ds(j * 16, 16)]
                vals = [plsc.load_gather(b_v, [s + (f * _N)]) for f in range(_FPW)]

                def cond(carry):
                    return carry[1] > 0

                def body(carry):
                    pend_i, _ = carry
                    pend = pend_i != 0
                    plsc.store_scatter(scr_v, [d], lane, mask=pend)
                    win = plsc.load_gather(scr_v, [d])
                    w = pend & (win == lane)
                    for f in range(_FPW):
                        df = d + (f * _N)
                        cur = plsc.load_gather(m_v, [df])
                        plsc.store_scatter(m_v, [df],
                                           jnp.maximum(cur, vals[f]), mask=w)
                    npend_i = jnp.where(w, jnp.zeros_like(pend_i), pend_i)
                    return npend_i, jnp.sum(npend_i)

                lax.while_loop(cond, body,
                               (jnp.ones((16,), jnp.int32), jnp.int32(16)))
                return c
            lax.fori_loop(0, _GROUPS, group, 0)

        def start(c, sb, db, ss, sd):
            pltpu.make_async_copy(src_hbm.at[pl.ds(c * _CHUNK, _CHUNK)], sb, ss).start()
            pltpu.make_async_copy(dst_hbm.at[pl.ds(c * _CHUNK, _CHUNK)], db, sd).start()

        def wait(sb, db, ss, sd):
            pltpu.make_async_copy(src_hbm.at[pl.ds(0, _CHUNK)], sb, ss).wait()
            pltpu.make_async_copy(dst_hbm.at[pl.ds(0, _CHUNK)], db, sd).wait()

        start(0, s0, d0, sem_s0, sem_d0)

        def pair(i, c):
            start(2 * i + 1, s1, d1, sem_s1, sem_d1)
            wait(s0, d0, sem_s0, sem_d0)
            process(s0, d0)

            @pl.when(i < _NCHUNKS // 2 - 1)
            def _():
                start(2 * i + 2, s0, d0, sem_s0, sem_d0)

            wait(s1, d1, sem_s1, sem_d1)
            process(s1, d1)
            return c
        lax.fori_loop(0, _NCHUNKS // 2, pair, 0)

        pltpu.sync_copy(m_v, out_hbm.at[pl.ds(base, _FPW * _N)])

    return jnp.reshape(k(jnp.reshape(bt_flat, (-1,)), src, dst), (_D, _N))


def kernel(x, edge_index, W1, b1, W2, b2):
    src = edge_index[0]
    dst = edge_index[1]
    b1col = jnp.reshape(b1, (_D, 1))
    b2col = jnp.reshape(b2, (_D, 1))

    a1t, b1t = _layer1_matmuls(x, W1, b1col)
    m1t = _sc_segment_max(b1t, src, dst)
    a2t, b2t = _layer2_matmuls(a1t, m1t, W2, b2col)
    m2t = _sc_segment_max(b2t, src, dst)
    return _finalize(a2t, m2t)
